# baseline (speedup 1.0000x reference)
import functools

import jax
import jax.numpy as jnp
from jax import lax
from jax.experimental import pallas as pl
from jax.experimental.pallas import tpu as pltpu
from jax.experimental.pallas import tpu_sc as plsc

NCLS = 19
SMOOTH_V = 1e-05

NC = 2
NS = 16
L = 16
NW = NC * NS

TOTAL = 4 * 512 * 512
PER_W = TOTAL // NW
ACC = 2 * NCLS * L


def _sc_body(o_hbm, g_hbm, part_hbm, o_v, g_v, acc_v):
    wid = lax.axis_index("s") * NC + lax.axis_index("c")
    base = wid * PER_W
    pltpu.sync_copy(o_hbm.at[pl.ds(base, PER_W)], o_v)
    pltpu.sync_copy(g_hbm.at[pl.ds(base, PER_W)], g_v)

    zeros = jnp.zeros((L,), jnp.float32)
    for r in range(ACC // L):
        acc_v[pl.ds(r * L, L)] = zeros

    lane = lax.iota(jnp.int32, L)
    ones = jnp.ones((L,), jnp.float32)

    def body(i, carry):
        g = g_v[pl.ds(i * L, L)]
        o = o_v[pl.ds(i * L, L)]
        d = g.astype(jnp.float32) - o
        d2 = d * d
        idx = g * L + lane
        plsc.addupdate_scatter(acc_v, [idx], d2)
        plsc.addupdate_scatter(acc_v, [idx + NCLS * L], ones)
        return carry

    lax.fori_loop(0, PER_W // L, body, 0, unroll=8)
    pltpu.sync_copy(acc_v, part_hbm.at[wid])


@functools.partial(jax.jit)
def _sc_call(o_flat, g_flat):
    k = functools.partial(
        pl.kernel,
        out_type=jax.ShapeDtypeStruct((NW, ACC), jnp.float32),
        mesh=plsc.VectorSubcoreMesh(core_axis_name="c", subcore_axis_name="s"),
        compiler_params=pltpu.CompilerParams(needs_layout_passes=False),
        scratch_types=[
            pltpu.VMEM((PER_W,), jnp.float32),
            pltpu.VMEM((PER_W,), jnp.int32),
            pltpu.VMEM((ACC,), jnp.float32),
        ],
    )(_sc_body)
    return k(o_flat, g_flat)


def kernel(outputs, gt):
    o_flat = outputs.reshape(-1)
    g_flat = gt.reshape(-1)
    part = _sc_call(o_flat, g_flat)
    total = part.sum(axis=0)
    d2 = total[: NCLS * L].reshape(NCLS, L).sum(axis=-1)
    cnt = total[NCLS * L :].reshape(NCLS, L).sum(axis=-1)
    return d2 / jnp.maximum(cnt, SMOOTH_V)


# speedup vs baseline: 1.1068x; 1.1068x over previous
import functools

import jax
import jax.numpy as jnp
from jax import lax
from jax.experimental import pallas as pl
from jax.experimental.pallas import tpu as pltpu
from jax.experimental.pallas import tpu_sc as plsc

NCLS = 19
SMOOTH_V = 1e-05

NC = 2
NS = 16
L = 16
NW = NC * NS

B, H, W = 4, 512, 512
ROWS_PER_W = (B * H) // NW
NACC = 4
ACC = NCLS * L


def _sc_body(o_hbm, g_hbm, part_hbm, o_v, g_v, *accs):
    acc_a = accs[:NACC]
    acc_b = accs[NACC:]
    wid = lax.axis_index("s") * NC + lax.axis_index("c")
    b = wid // (NW // B)
    r0 = (wid % (NW // B)) * ROWS_PER_W
    pltpu.sync_copy(o_hbm.at[b, 0, pl.ds(r0, ROWS_PER_W), :], o_v)
    pltpu.sync_copy(g_hbm.at[b, 0, pl.ds(r0, ROWS_PER_W), :], g_v)

    zeros = jnp.zeros((L,), jnp.float32)
    for a in accs:
        for r in range(ACC // L):
            a[pl.ds(r * L, L)] = zeros

    lane = lax.iota(jnp.int32, L)
    ones = jnp.ones((L,), jnp.float32)

    def row_body(r, carry):
        for j in range(W // L):
            g = g_v[r, pl.ds(j * L, L)]
            o = o_v[r, pl.ds(j * L, L)]
            d = g.astype(jnp.float32) - o
            d2 = d * d
            idx = g * L + lane
            plsc.addupdate_scatter(acc_a[j % NACC], [idx], d2)
            plsc.addupdate_scatter(acc_b[j % NACC], [idx], ones)
        return carry

    lax.fori_loop(0, ROWS_PER_W, row_body, 0)

    for r in range(ACC // L):
        sl = pl.ds(r * L, L)
        acc_a[0][sl] += acc_a[1][sl] + acc_a[2][sl] + acc_a[3][sl]
        acc_b[0][sl] += acc_b[1][sl] + acc_b[2][sl] + acc_b[3][sl]
    pltpu.sync_copy(acc_a[0], part_hbm.at[wid, 0])
    pltpu.sync_copy(acc_b[0], part_hbm.at[wid, 1])


@jax.jit
def _sc_call(o, g):
    k = functools.partial(
        pl.kernel,
        out_type=jax.ShapeDtypeStruct((NW, 2, ACC), jnp.float32),
        mesh=plsc.VectorSubcoreMesh(core_axis_name="c", subcore_axis_name="s"),
        compiler_params=pltpu.CompilerParams(needs_layout_passes=False),
        scratch_types=[
            pltpu.VMEM((ROWS_PER_W, W), jnp.float32),
            pltpu.VMEM((ROWS_PER_W, W), jnp.int32),
        ]
        + [pltpu.VMEM((ACC,), jnp.float32) for _ in range(2 * NACC)],
    )(_sc_body)
    return k(o, g)


def kernel(outputs, gt):
    part = _sc_call(outputs, gt)
    total = part.sum(axis=0)
    d2 = total[0].reshape(NCLS, L).sum(axis=-1)
    cnt = total[1].reshape(NCLS, L).sum(axis=-1)
    return d2 / jnp.maximum(cnt, SMOOTH_V)


# speedup vs baseline: 1.5486x; 1.3992x over previous
import functools

import jax
import jax.numpy as jnp
from jax import lax
from jax.experimental import pallas as pl
from jax.experimental.pallas import tpu as pltpu
from jax.experimental.pallas import tpu_sc as plsc

NCLS = 19
SMOOTH_V = 1e-05

NC = 2
NS = 16
L = 16
NW = NC * NS

B, H, W = 4, 512, 512
ROWS_PER_W = (B * H) // NW
CHUNK = 16
NCHUNK = ROWS_PER_W // CHUNK
GRP = 8
NACC = 4
ACC = NCLS * L


def _sc_body(o_hbm, g_hbm, part_hbm, o_v, g_v, sem_o, sem_g, *accs):
    acc_a = accs[:NACC]
    acc_b = accs[NACC:]
    wid = lax.axis_index("s") * NC + lax.axis_index("c")
    b = wid // (NW // B)
    r0 = (wid % (NW // B)) * ROWS_PER_W

    def start(c, slot):
        ro = r0 + c * CHUNK
        co = pltpu.async_copy(
            o_hbm.at[b, 0, pl.ds(ro, CHUNK), :], o_v.at[slot], sem_o.at[slot]
        )
        cg = pltpu.async_copy(
            g_hbm.at[b, 0, pl.ds(ro, CHUNK), :], g_v.at[slot], sem_g.at[slot]
        )
        return co, cg

    pend = start(0, 0)

    zeros = jnp.zeros((L,), jnp.float32)
    for a in accs:
        for r in range(ACC // L):
            a[pl.ds(r * L, L)] = zeros

    lane = lax.iota(jnp.int32, L)
    ones = jnp.ones((L,), jnp.float32)

    def do_chunk(slot):
        def row_body(r, carry):
            for grp in range(W // (L * GRP)):
                gs, os_ = [], []
                for j in range(GRP):
                    col = (grp * GRP + j) * L
                    gs.append(g_v[slot, r, pl.ds(col, L)])
                    os_.append(o_v[slot, r, pl.ds(col, L)])
                d2s, idxs = [], []
                for j in range(GRP):
                    d = gs[j].astype(jnp.float32) - os_[j]
                    d2s.append(d * d)
                    idxs.append(gs[j] * L + lane)
                for j in range(GRP):
                    plsc.addupdate_scatter(acc_a[j % NACC], [idxs[j]], d2s[j])
                    plsc.addupdate_scatter(acc_b[j % NACC], [idxs[j]], ones)
            return carry

        lax.fori_loop(0, CHUNK, row_body, 0)

    for c in range(NCHUNK):
        co, cg = pend
        co.wait()
        cg.wait()
        if c + 1 < NCHUNK:
            pend = start(c + 1, (c + 1) % 2)
        do_chunk(c % 2)

    for r in range(ACC // L):
        sl = pl.ds(r * L, L)
        acc_a[0][sl] += acc_a[1][sl] + acc_a[2][sl] + acc_a[3][sl]
        acc_b[0][sl] += acc_b[1][sl] + acc_b[2][sl] + acc_b[3][sl]
    pltpu.sync_copy(acc_a[0], part_hbm.at[wid, 0])
    pltpu.sync_copy(acc_b[0], part_hbm.at[wid, 1])


@jax.jit
def _sc_call(o, g):
    k = functools.partial(
        pl.kernel,
        out_type=jax.ShapeDtypeStruct((NW, 2, ACC), jnp.float32),
        mesh=plsc.VectorSubcoreMesh(core_axis_name="c", subcore_axis_name="s"),
        compiler_params=pltpu.CompilerParams(needs_layout_passes=False),
        scratch_types=[
            pltpu.VMEM((2, CHUNK, W), jnp.float32),
            pltpu.VMEM((2, CHUNK, W), jnp.int32),
            pltpu.SemaphoreType.DMA((2,)),
            pltpu.SemaphoreType.DMA((2,)),
        ]
        + [pltpu.VMEM((ACC,), jnp.float32) for _ in range(2 * NACC)],
    )(_sc_body)
    return k(o, g)


def kernel(outputs, gt):
    part = _sc_call(outputs, gt)
    total = part.sum(axis=0)
    d2 = total[0].reshape(NCLS, L).sum(axis=-1)
    cnt = total[1].reshape(NCLS, L).sum(axis=-1)
    return d2 / jnp.maximum(cnt, SMOOTH_V)
